# Initial kernel scaffold; baseline (speedup 1.0000x reference)
import jax
import jax.numpy as jnp
from jax.experimental import pallas as pl


def kernel(x, edge_index, knn_graph, W_s1, b_s1, W_s2, b_s2, W_f2, b_f2):
    raise NotImplementedError("write your pallas kernel here")


# speedup vs baseline: 5.5010x; 5.5010x over previous
import functools

import jax
import jax.numpy as jnp
from jax import lax
from jax.experimental import pallas as pl
from jax.experimental.pallas import tpu as pltpu
from jax.experimental.pallas import tpu_sc as plsc

N = 10000
D = 128
NC = 2
NS = 16
L = 16
R = 10240
DUM = 10000
WS = D // NC
FCH = WS // L
CH = 128
RPT = R // NS
CCH = 40
ZCH = 64

EM = 320000
EK = 160000
ROWS_M = NS * ((EM + NS * CH - 1) // (NS * CH))
ROWS_K = NS * ((EK + NS * CH - 1) // (NS * CH))
CPT_M = ROWS_M // NS
CPT_K = ROWS_K // NS

@functools.cache
def _mesh():
  return plsc.VectorSubcoreMesh(
      core_axis_name="c", subcore_axis_name="s", num_cores=NC, num_subcores=NS)


def _fill(buf, rows, width, value):
  v = jnp.full((L,), value, jnp.float32)
  for i in range(rows):
    for f in range(width // L):
      buf[i, pl.ds(f * L, L)] = v


def _zero_acc(acc, zbuf, sub, width):
  z0 = sub * (ZCH * (R // (NS * ZCH)))
  for i in range(R // (NS * ZCH)):
    pltpu.sync_copy(zbuf, acc.at[pl.ds(z0 + i * ZCH, ZCH)])


def _scatter_edges(u_ref, src_ref, dst_ref, cpt, acc, sbuf, dbuf, gidx, sidx,
                   rbuf, base, sub):
  ebase = sub * cpt * CH

  def body(i, carry):
    off = ebase + i * CH
    pltpu.sync_copy(src_ref.at[pl.ds(off, CH)], sbuf)
    pltpu.sync_copy(dst_ref.at[pl.ds(off, CH)], dbuf)
    for j in range(CH // L):
      s = sbuf[pl.ds(j * L, L)]
      dd = dbuf[pl.ds(j * L, L)]
      gidx[pl.ds(j * L, L)] = s + base
      sidx[pl.ds(j * L, L)] = jnp.where(s == dd, jnp.int32(DUM), dd)
    pltpu.sync_copy(u_ref.at[gidx], rbuf)
    pltpu.sync_copy(rbuf, acc.at[sidx], add=True)
    return carry

  lax.fori_loop(0, cpt, body, 0)


def _combine(src_ref, dst_ref, scale_ref, z_ref, acc, abuf, ubuf, obuf, scbuf,
             zcbuf, base, sub, width):
  nrow0 = sub * RPT

  def body(i, carry):
    r0 = nrow0 + i * CCH
    pltpu.sync_copy(acc.at[pl.ds(r0, CCH)], abuf)
    pltpu.sync_copy(src_ref.at[pl.ds(base + r0, CCH)], ubuf)
    pltpu.sync_copy(scale_ref.at[pl.ds(r0, CCH)], scbuf)
    if z_ref is not None:
      pltpu.sync_copy(z_ref.at[pl.ds(base + r0, CCH)], zcbuf)

    def rbody(r, c):
      sc = scbuf[r, :]
      for f in range(width // L):
        av = abuf[r, pl.ds(f * L, L)]
        uv = ubuf[r, pl.ds(f * L, L)]
        ov = sc * (av + uv)
        if z_ref is not None:
          ov = ov + zcbuf[r, pl.ds(f * L, L)]
        obuf[r, pl.ds(f * L, L)] = ov
      return c

    lax.fori_loop(0, CCH, rbody, 0)
    pltpu.sync_copy(obuf, dst_ref.at[pl.ds(base + r0, CCH)])
    return carry

  lax.fori_loop(0, RPT // CCH, body, 0)


def _make_chain(K, cpt, with_z):
  def body(u0, src_e, dst_e, d2, dinv, *rest):
    if with_z:
      z0, outa, outb = rest[0], rest[1], rest[2]
      scratches = rest[3:]
    else:
      z0 = None
      outa, outb = rest[0], rest[1]
      scratches = rest[2:]
    acc, sbuf, dbuf, gidx, sidx, rbuf, abuf, ubuf, obuf, scbuf, zcbuf, zbuf = \
        scratches
    core = lax.axis_index("c")
    sub = lax.axis_index("s")
    base = core * R
    _fill(zbuf, ZCH, WS, 0.0)

    def step(s_ref, t_ref, scale_ref, zr=None):
      _zero_acc(acc, zbuf, sub, WS)
      plsc.subcore_barrier()
      _scatter_edges(s_ref, src_e, dst_e, cpt, acc, sbuf, dbuf, gidx, sidx,
                     rbuf, base, sub)
      plsc.subcore_barrier()
      _combine(s_ref, t_ref, scale_ref, zr, acc, abuf, ubuf, obuf, scbuf,
               zcbuf, base, sub, WS)
      plsc.subcore_barrier()

    if K == 1:
      step(u0, outa, dinv, z0)
    elif K == 2:
      step(u0, outa, d2)
      step(outa, outb, dinv)
    else:
      assert K % 2 == 0 and K >= 4
      step(u0, outa, d2)

      def dbl(i, c):
        step(outa, outb, d2)
        step(outb, outa, d2)
        return c

      lax.fori_loop(0, (K - 2) // 2, dbl, 0)
      step(outa, outb, dinv)

  out_t = (jax.ShapeDtypeStruct((NC * R, WS), jnp.float32),
           jax.ShapeDtypeStruct((NC * R, WS), jnp.float32))
  scratch = [
      pltpu.VMEM_SHARED((R, WS), jnp.float32),
      pltpu.VMEM((CH,), jnp.int32),
      pltpu.VMEM((CH,), jnp.int32),
      pltpu.VMEM((CH,), jnp.int32),
      pltpu.VMEM((CH,), jnp.int32),
      pltpu.VMEM((CH, WS), jnp.float32),
      pltpu.VMEM((CCH, WS), jnp.float32),
      pltpu.VMEM((CCH, WS), jnp.float32),
      pltpu.VMEM((CCH, WS), jnp.float32),
      pltpu.VMEM((CCH, L), jnp.float32),
      pltpu.VMEM((CCH, WS), jnp.float32),
      pltpu.VMEM((ZCH, WS), jnp.float32),
  ]
  return pl.kernel(body, out_type=out_t, mesh=_mesh(), scratch_types=scratch,
                   compiler_params=pltpu.CompilerParams(
                       use_tc_tiling_on_sc=False))


def _deg_body(src_m, dst_m, src_k, dst_k, deg_m, deg_k,
              acc, sbuf, dbuf, sidx, onesb, abuf, zbuf):
  core = lax.axis_index("c")
  sub = lax.axis_index("s")
  _fill(zbuf, ZCH, L, 0.0)
  _fill(onesb, CH, L, 1.0)

  def histo(src_ref, dst_ref, cpt, deg_out):
    _zero_acc(acc, zbuf, sub, L)
    plsc.subcore_barrier()
    ebase = sub * cpt * CH

    def body(i, c):
      off = ebase + i * CH
      pltpu.sync_copy(src_ref.at[pl.ds(off, CH)], sbuf)
      pltpu.sync_copy(dst_ref.at[pl.ds(off, CH)], dbuf)
      for j in range(CH // L):
        s = sbuf[pl.ds(j * L, L)]
        dd = dbuf[pl.ds(j * L, L)]
        sidx[pl.ds(j * L, L)] = jnp.where(s == dd, jnp.int32(DUM), dd)
      pltpu.sync_copy(onesb, acc.at[sidx], add=True)
      return c

    lax.fori_loop(0, cpt, body, 0)
    plsc.subcore_barrier()
    nrow0 = sub * RPT

    def comb(i, c):
      r0 = nrow0 + i * CCH
      pltpu.sync_copy(acc.at[pl.ds(r0, CCH)], abuf)

      def rbody(r, cc):
        abuf[r, :] = abuf[r, :] + 1.0
        return cc

      lax.fori_loop(0, CCH, rbody, 0)
      pltpu.sync_copy(abuf, deg_out.at[pl.ds(r0, CCH)])
      return c

    lax.fori_loop(0, RPT // CCH, comb, 0)

  @pl.when(core == 0)
  def _():
    histo(src_m, dst_m, CPT_M, deg_m)

  @pl.when(core == 1)
  def _():
    histo(src_k, dst_k, CPT_K, deg_k)


@functools.cache
def _deg_kernel():
  return pl.kernel(
    _deg_body,
    out_type=(jax.ShapeDtypeStruct((R, L), jnp.float32),) * 2,
    mesh=_mesh(),
    scratch_types=[
        pltpu.VMEM_SHARED((R, L), jnp.float32),
        pltpu.VMEM((CH,), jnp.int32),
        pltpu.VMEM((CH,), jnp.int32),
        pltpu.VMEM((CH,), jnp.int32),
        pltpu.VMEM((CH, L), jnp.float32),
        pltpu.VMEM((CCH, L), jnp.float32),
        pltpu.VMEM((ZCH, L), jnp.float32),
    ],
    compiler_params=pltpu.CompilerParams(use_tc_tiling_on_sc=False),
  )


def _norm_body(dm_ref, dk_ref, im_ref, qm_ref, ik_ref, qk_ref):
  dm = dm_ref[...]
  dk = dk_ref[...]
  im_ref[...] = lax.rsqrt(dm)
  qm_ref[...] = 1.0 / dm
  ik_ref[...] = lax.rsqrt(dk)
  qk_ref[...] = 1.0 / dk


def _norm(deg_m, deg_k):
  return pl.pallas_call(
      _norm_body,
      out_shape=(jax.ShapeDtypeStruct((R, L), jnp.float32),) * 4,
  )(deg_m, deg_k)


def _mm_body(x_ref, w_ref, b_ref, dm_ref, dk_ref, o_ref):
  y = jnp.dot(x_ref[...], w_ref[...], preferred_element_type=jnp.float32)
  y = y + b_ref[0:1, :]
  s = dm_ref[:, 0:1]
  sk = dk_ref[:, 0:1]
  o_ref[:, 0:384] = y[:, 0:384] * s
  o_ref[:, 384:512] = y[:, 384:512] * sk


_MMB = 1000


def _matmul(x, wt, b8, dm, dk):
  return pl.pallas_call(
      _mm_body,
      grid=(N // _MMB,),
      in_specs=[
          pl.BlockSpec((_MMB, D), lambda i: (i, 0)),
          pl.BlockSpec((D, 512), lambda i: (0, 0)),
          pl.BlockSpec((8, 512), lambda i: (0, 0)),
          pl.BlockSpec((_MMB, L), lambda i: (i, 0)),
          pl.BlockSpec((_MMB, L), lambda i: (i, 0)),
      ],
      out_specs=pl.BlockSpec((_MMB, 512), lambda i: (i, 0)),
      out_shape=jax.ShapeDtypeStruct((N, 512), jnp.float32),
  )(x, wt, b8, dm, dk)


def _pad_edges(ei, rows):
  e = ei.shape[1]
  tot = rows * CH
  src = jnp.pad(ei[0].astype(jnp.int32), (0, tot - e))
  pad_d = DUM + (jnp.arange(tot - e, dtype=jnp.int32) % (R - DUM))
  dst = jnp.concatenate([ei[1].astype(jnp.int32), pad_d])
  return src, dst


def _to_sc(y):
  lo = jnp.pad(y[:, 0:WS], ((0, R - N), (0, 0)))
  hi = jnp.pad(y[:, WS:D], ((0, R - N), (0, 0)))
  return jnp.concatenate([lo, hi], axis=0)


def _from_sc(u):
  return jnp.concatenate([u[0:N], u[R:R + N]], axis=1)


_chain2 = functools.cache(lambda: _make_chain(2, CPT_M, False))
_chain10 = functools.cache(lambda: _make_chain(10, CPT_M, False))
_chain1z = functools.cache(lambda: _make_chain(1, CPT_K, True))


def kernel(x, edge_index, knn_graph, W_s1, b_s1, W_s2, b_s2, W_f2, b_f2):
  src_m, dst_m = _pad_edges(edge_index, ROWS_M)
  src_k, dst_k = _pad_edges(knn_graph, ROWS_K)

  deg_m, deg_k = _deg_kernel()(src_m, dst_m, src_k, dst_k)
  dinv_m, d2_m, dinv_k, d2_k = _norm(deg_m, deg_k)

  wt = jnp.concatenate([W_s1, W_s2, W_f2, W_f2], axis=0).T
  bcat = jnp.concatenate([b_s1, b_s2, b_f2, b_f2])
  b8 = jnp.broadcast_to(bcat[None, :], (8, 512))
  y = _matmul(x, wt, b8, dinv_m[0:N], dinv_k[0:N])

  u_s1 = _to_sc(y[:, 0:128])
  u_s2 = _to_sc(y[:, 128:256])
  u_f = _to_sc(y[:, 256:384])
  u_k = _to_sc(y[:, 384:512])

  _, h0u = _chain2()(u_s1, src_m, dst_m, d2_m, dinv_m)
  _, h1u = _chain10()(u_s2, src_m, dst_m, d2_m, dinv_m)
  _, z0u = _chain2()(u_f, src_m, dst_m, d2_m, dinv_m)
  z1u, _ = _chain1z()(u_k, src_k, dst_k, d2_k, dinv_k, z0u)

  return (_from_sc(h0u), _from_sc(h1u), _from_sc(z0u), _from_sc(z1u))


# speedup vs baseline: 6.8321x; 1.2420x over previous
import functools

import jax
import jax.numpy as jnp
from jax import lax
from jax.experimental import pallas as pl
from jax.experimental.pallas import tpu as pltpu
from jax.experimental.pallas import tpu_sc as plsc

N = 10000
D = 128
NC = 2
NS = 16
L = 16
R = 10240
DUM = 10000
WS = D // NC
CH = 128
RPT = R // NS
CCH = 40
ZCH = 64

EM = 320000
EK = 160000
CPT_M = 8 * ((EM + NS * CH * 8 - 1) // (NS * CH * 8))
CPT_K = 8 * ((EK + NS * CH * 8 - 1) // (NS * CH * 8))
ROWS_M = NS * CPT_M
ROWS_K = NS * CPT_K


@functools.cache
def _mesh():
  return plsc.VectorSubcoreMesh(
      core_axis_name="c", subcore_axis_name="s", num_cores=NC, num_subcores=NS)


def _fill(buf, rows, width, value):
  v = jnp.full((L,), value, jnp.float32)
  for i in range(rows):
    for f in range(width // L):
      buf[i, pl.ds(f * L, L)] = v


def _zero_acc(acc, zbuf, sub):
  z0 = sub * RPT
  for i in range(RPT // ZCH):
    pltpu.sync_copy(zbuf, acc.at[pl.ds(z0 + i * ZCH, ZCH)])


def _stage_indices(src_ref, dst_ref, cpt, sball, dball, base, sub):
  pltpu.sync_copy(src_ref.at[pl.ds(sub * cpt, cpt)], sball.at[pl.ds(0, cpt)])
  pltpu.sync_copy(dst_ref.at[pl.ds(sub * cpt, cpt)], dball.at[pl.ds(0, cpt)])

  def body(i, carry):
    for j in range(CH // L):
      s = sball[i, pl.ds(j * L, L)]
      dd = dball[i, pl.ds(j * L, L)]
      sball[i, pl.ds(j * L, L)] = s + base
      dball[i, pl.ds(j * L, L)] = jnp.where(s == dd, jnp.int32(DUM), dd)
    return carry

  lax.fori_loop(0, cpt, body, 0)


def _scatter_step(u_ref, cpt, acc, sball, dball, rb0, rb1, sem0, sem1):
  def g(c, rb, sem):
    pltpu.async_copy(u_ref.at[sball.at[c]], rb, sem)

  def gw(rb, sem):
    pltpu.make_async_copy(u_ref.at[pl.ds(0, CH)], rb, sem).wait()

  def s(c, rb):
    pltpu.sync_copy(rb, acc.at[dball.at[c]], add=True)

  g(0, rb0, sem0)
  g(1, rb1, sem1)

  def body(i, carry):
    c = 2 * i
    gw(rb0, sem0)
    s(c, rb0)
    g(c + 2, rb0, sem0)
    gw(rb1, sem1)
    s(c + 1, rb1)
    g(c + 3, rb1, sem1)
    return carry

  lax.fori_loop(0, (cpt - 2) // 2, body, 0)
  gw(rb0, sem0)
  s(cpt - 2, rb0)
  gw(rb1, sem1)
  s(cpt - 1, rb1)


def _combine(src_ref, dst_ref, z_ref, acc, abuf, ubuf, obuf, scbuf, zcbuf,
             base, sub, scale_lo):
  nrow0 = sub * RPT

  def body(i, carry):
    r0 = nrow0 + i * CCH
    pltpu.sync_copy(acc.at[pl.ds(r0, CCH)], abuf)
    pltpu.sync_copy(src_ref.at[pl.ds(base + r0, CCH)], ubuf)
    if z_ref is not None:
      pltpu.sync_copy(z_ref.at[pl.ds(base + r0, CCH)], zcbuf)

    def rbody(r, c):
      sc = scbuf[scale_lo + i * CCH + r, :]
      for f in range(WS // L):
        ov = sc * (abuf[r, pl.ds(f * L, L)] + ubuf[r, pl.ds(f * L, L)])
        if z_ref is not None:
          ov = ov + zcbuf[r, pl.ds(f * L, L)]
        obuf[r, pl.ds(f * L, L)] = ov
      return c

    lax.fori_loop(0, CCH, rbody, 0)
    pltpu.sync_copy(obuf, dst_ref.at[pl.ds(base + r0, CCH)])
    return carry

  lax.fori_loop(0, RPT // CCH, body, 0)


def _make_chain(K, cpt, with_z):
  def body(u0, src_e, dst_e, d2, dinv, *rest):
    if with_z:
      z0, outa, outb = rest[0], rest[1], rest[2]
      scratches = rest[3:]
    else:
      z0 = None
      outa, outb = rest[0], rest[1]
      scratches = rest[2:]
    (acc, sball, dball, rb0, rb1, abuf, ubuf, obuf, scbuf, zcbuf, zbuf,
     sem0, sem1) = scratches
    core = lax.axis_index("c")
    sub = lax.axis_index("s")
    base = core * R
    _fill(zbuf, ZCH, WS, 0.0)
    _stage_indices(src_e, dst_e, cpt, sball, dball, base, sub)
    nrow0 = sub * RPT
    pltpu.sync_copy(d2.at[pl.ds(nrow0, RPT)], scbuf.at[pl.ds(0, RPT)])
    pltpu.sync_copy(dinv.at[pl.ds(nrow0, RPT)], scbuf.at[pl.ds(RPT, RPT)])

    def step(s_ref, t_ref, last, zr=None):
      _zero_acc(acc, zbuf, sub)
      plsc.subcore_barrier()
      _scatter_step(s_ref, cpt, acc, sball, dball, rb0, rb1, sem0, sem1)
      plsc.subcore_barrier()
      _combine(s_ref, t_ref, zr, acc, abuf, ubuf, obuf, scbuf, zcbuf,
               base, sub, RPT if last else 0)
      plsc.subcore_barrier()

    if K == 1:
      step(u0, outa, True, z0)
    elif K == 2:
      step(u0, outa, False)
      step(outa, outb, True)
    else:
      assert K % 2 == 0 and K >= 4
      step(u0, outa, False)

      def dbl(i, c):
        step(outa, outb, False)
        step(outb, outa, False)
        return c

      lax.fori_loop(0, (K - 2) // 2, dbl, 0)
      step(outa, outb, True)

  out_t = (jax.ShapeDtypeStruct((NC * R, WS), jnp.float32),
           jax.ShapeDtypeStruct((NC * R, WS), jnp.float32))
  scratch = [
      pltpu.VMEM_SHARED((R, WS), jnp.float32),
      pltpu.VMEM((cpt, CH), jnp.int32),
      pltpu.VMEM((cpt, CH), jnp.int32),
      pltpu.VMEM((CH, WS), jnp.float32),
      pltpu.VMEM((CH, WS), jnp.float32),
      pltpu.VMEM((CCH, WS), jnp.float32),
      pltpu.VMEM((CCH, WS), jnp.float32),
      pltpu.VMEM((CCH, WS), jnp.float32),
      pltpu.VMEM((2 * RPT, L), jnp.float32),
      pltpu.VMEM((CCH, WS), jnp.float32),
      pltpu.VMEM((ZCH, WS), jnp.float32),
      pltpu.SemaphoreType.DMA,
      pltpu.SemaphoreType.DMA,
  ]
  return pl.kernel(body, out_type=out_t, mesh=_mesh(), scratch_types=scratch,
                   compiler_params=pltpu.CompilerParams(
                       use_tc_tiling_on_sc=False))


def _deg_body(src_m, dst_m, src_k, dst_k, deg_m, deg_k,
              acc, sball, dball, onesb, abuf, zbuf, sem):
  core = lax.axis_index("c")
  sub = lax.axis_index("s")
  _fill(zbuf, ZCH, L, 0.0)
  _fill(onesb, CH, L, 1.0)

  def histo(src_ref, dst_ref, cpt, deg_out):
    _stage_indices(src_ref, dst_ref, cpt, sball, dball, 0, sub)
    _zero_acc(acc, zbuf, sub)
    plsc.subcore_barrier()

    def group(i, c):
      for j in range(8):
        pltpu.async_copy(onesb, acc.at[dball.at[8 * i + j]], sem, add=True)
      for j in range(8):
        pltpu.make_async_copy(deg_out.at[pl.ds(0, CH)], onesb, sem).wait()
      return c

    lax.fori_loop(0, cpt // 8, group, 0)
    plsc.subcore_barrier()
    nrow0 = sub * RPT

    def comb(i, c):
      r0 = nrow0 + i * CCH
      pltpu.sync_copy(acc.at[pl.ds(r0, CCH)], abuf)

      def rbody(r, cc):
        abuf[r, :] = abuf[r, :] + 1.0
        return cc

      lax.fori_loop(0, CCH, rbody, 0)
      pltpu.sync_copy(abuf, deg_out.at[pl.ds(r0, CCH)])
      return c

    lax.fori_loop(0, RPT // CCH, comb, 0)

  @pl.when(core == 0)
  def _():
    histo(src_m, dst_m, CPT_M, deg_m)

  @pl.when(core == 1)
  def _():
    histo(src_k, dst_k, CPT_K, deg_k)


@functools.cache
def _deg_kernel():
  return pl.kernel(
    _deg_body,
    out_type=(jax.ShapeDtypeStruct((R, L), jnp.float32),) * 2,
    mesh=_mesh(),
    scratch_types=[
        pltpu.VMEM_SHARED((R, L), jnp.float32),
        pltpu.VMEM((CPT_M, CH), jnp.int32),
        pltpu.VMEM((CPT_M, CH), jnp.int32),
        pltpu.VMEM((CH, L), jnp.float32),
        pltpu.VMEM((CCH, L), jnp.float32),
        pltpu.VMEM((ZCH, L), jnp.float32),
        pltpu.SemaphoreType.DMA,
    ],
    compiler_params=pltpu.CompilerParams(use_tc_tiling_on_sc=False),
  )


def _norm_body(dm_ref, dk_ref, im_ref, qm_ref, ik_ref, qk_ref):
  dm = dm_ref[...]
  dk = dk_ref[...]
  im_ref[...] = lax.rsqrt(dm)
  qm_ref[...] = 1.0 / dm
  ik_ref[...] = lax.rsqrt(dk)
  qk_ref[...] = 1.0 / dk


def _norm(deg_m, deg_k):
  return pl.pallas_call(
      _norm_body,
      out_shape=(jax.ShapeDtypeStruct((R, L), jnp.float32),) * 4,
  )(deg_m, deg_k)


def _mm_body(x_ref, w_ref, b_ref, dm_ref, dk_ref, o_ref):
  y = jnp.dot(x_ref[...], w_ref[...], preferred_element_type=jnp.float32)
  y = y + b_ref[0:1, :]
  s = dm_ref[:, 0:1]
  sk = dk_ref[:, 0:1]
  o_ref[:, 0:384] = y[:, 0:384] * s
  o_ref[:, 384:512] = y[:, 384:512] * sk


_MMB = 1000


def _matmul(x, wt, b8, dm, dk):
  return pl.pallas_call(
      _mm_body,
      grid=(N // _MMB,),
      in_specs=[
          pl.BlockSpec((_MMB, D), lambda i: (i, 0)),
          pl.BlockSpec((D, 512), lambda i: (0, 0)),
          pl.BlockSpec((8, 512), lambda i: (0, 0)),
          pl.BlockSpec((_MMB, L), lambda i: (i, 0)),
          pl.BlockSpec((_MMB, L), lambda i: (i, 0)),
      ],
      out_specs=pl.BlockSpec((_MMB, 512), lambda i: (i, 0)),
      out_shape=jax.ShapeDtypeStruct((N, 512), jnp.float32),
  )(x, wt, b8, dm, dk)


def _pad_edges(ei, rows):
  e = ei.shape[1]
  tot = rows * CH
  src = jnp.pad(ei[0].astype(jnp.int32), (0, tot - e)).reshape(rows, CH)
  pad_d = DUM + (jnp.arange(tot - e, dtype=jnp.int32) % (R - DUM))
  dst = jnp.concatenate([ei[1].astype(jnp.int32), pad_d]).reshape(rows, CH)
  return src, dst


def _to_sc(y):
  lo = jnp.pad(y[:, 0:WS], ((0, R - N), (0, 0)))
  hi = jnp.pad(y[:, WS:D], ((0, R - N), (0, 0)))
  return jnp.concatenate([lo, hi], axis=0)


def _from_sc(u):
  return jnp.concatenate([u[0:N], u[R:R + N]], axis=1)


_chain2 = functools.cache(lambda: _make_chain(2, CPT_M, False))
_chain10 = functools.cache(lambda: _make_chain(10, CPT_M, False))
_chain1z = functools.cache(lambda: _make_chain(1, CPT_K, True))


def kernel(x, edge_index, knn_graph, W_s1, b_s1, W_s2, b_s2, W_f2, b_f2):
  src_m, dst_m = _pad_edges(edge_index, ROWS_M)
  src_k, dst_k = _pad_edges(knn_graph, ROWS_K)

  deg_m, deg_k = _deg_kernel()(src_m, dst_m, src_k, dst_k)
  dinv_m, d2_m, dinv_k, d2_k = _norm(deg_m, deg_k)

  wt = jnp.concatenate([W_s1, W_s2, W_f2, W_f2], axis=0).T
  bcat = jnp.concatenate([b_s1, b_s2, b_f2, b_f2])
  b8 = jnp.broadcast_to(bcat[None, :], (8, 512))
  y = _matmul(x, wt, b8, dinv_m[0:N], dinv_k[0:N])

  u_s1 = _to_sc(y[:, 0:128])
  u_s2 = _to_sc(y[:, 128:256])
  u_f = _to_sc(y[:, 256:384])
  u_k = _to_sc(y[:, 384:512])

  _, h0u = _chain2()(u_s1, src_m, dst_m, d2_m, dinv_m)
  _, h1u = _chain10()(u_s2, src_m, dst_m, d2_m, dinv_m)
  _, z0u = _chain2()(u_f, src_m, dst_m, d2_m, dinv_m)
  z1u, _ = _chain1z()(u_k, src_k, dst_k, d2_k, dinv_k, z0u)

  return (_from_sc(h0u), _from_sc(h1u), _from_sc(z0u), _from_sc(z1u))


# speedup vs baseline: 12.8588x; 1.8821x over previous
import functools

import jax
import jax.numpy as jnp
from jax import lax
from jax.experimental import pallas as pl
from jax.experimental.pallas import tpu as pltpu
from jax.experimental.pallas import tpu_sc as plsc

N = 10000
D = 128
NC = 2
NS = 16
L = 16
R = 10240
DUM = 10000
WS = D // NC
CH = 128
RPT = R // NS
CCH = 40

EM = 320000
EK = 160000
CPT_M = 8 * ((EM + NS * CH * 8 - 1) // (NS * CH * 8))
CPT_K = 8 * ((EK + NS * CH * 8 - 1) // (NS * CH * 8))
ROWS_M = NS * CPT_M
ROWS_K = NS * CPT_K


@functools.cache
def _mesh():
  return plsc.VectorSubcoreMesh(
      core_axis_name="c", subcore_axis_name="s", num_cores=NC, num_subcores=NS)


def _fill(buf, rows, width, value):
  v = jnp.full((L,), value, jnp.float32)
  for i in range(rows):
    for f in range(width // L):
      buf[i, pl.ds(f * L, L)] = v


def _chunk_idx(pball, c, gidx, sidx, base):
  for j in range(CH // L):
    w = pball[c, pl.ds(j * L, L)]
    s = w & jnp.int32(0xFFFF)
    dd = w >> 16
    gidx[pl.ds(j * L, L)] = s + base
    sidx[pl.ds(j * L, L)] = jnp.where(s == dd, jnp.int32(DUM), dd)


def _scatter_step(u_ref, cpt, acc, pball, gidx0, sidx0, gidx1, sidx1,
                  rb0, rb1, sem0, sem1, base, dummy_hbm):
  def g(c, gidx, sidx, rb, sem):
    _chunk_idx(pball, c, gidx, sidx, base)
    pltpu.async_copy(u_ref.at[gidx], rb, sem)

  def gw(rb, sem):
    pltpu.make_async_copy(dummy_hbm, rb, sem).wait()

  def s(sidx, rb):
    pltpu.sync_copy(rb, acc.at[sidx], add=True)

  g(0, gidx0, sidx0, rb0, sem0)
  g(1, gidx1, sidx1, rb1, sem1)

  def body(i, carry):
    c = 2 * i
    gw(rb0, sem0)
    s(sidx0, rb0)
    g(c + 2, gidx0, sidx0, rb0, sem0)
    gw(rb1, sem1)
    s(sidx1, rb1)
    g(c + 3, gidx1, sidx1, rb1, sem1)
    return carry

  lax.fori_loop(0, (cpt - 2) // 2, body, 0)
  gw(rb0, sem0)
  s(sidx0, rb0)
  gw(rb1, sem1)
  s(sidx1, rb1)


def _combine(src_ref, dst_ref, z_ref, scale_ref, acc, abuf, ubuf, obuf, scb,
             zcbuf, zbuf, src_off, dst_off, sub, rezero):
  nrow0 = sub * RPT

  def body(i, carry):
    r0 = nrow0 + i * CCH
    pltpu.sync_copy(acc.at[pl.ds(r0, CCH)], abuf)
    pltpu.sync_copy(src_ref.at[pl.ds(src_off + r0, CCH)], ubuf)
    pltpu.sync_copy(scale_ref.at[pl.ds(r0, CCH)], scb)
    if rezero:
      pltpu.sync_copy(zbuf, acc.at[pl.ds(r0, CCH)])
    if z_ref is not None:
      pltpu.sync_copy(z_ref.at[pl.ds(dst_off + r0, CCH)], zcbuf)

    def rbody(r, c):
      sc = scb[r, :]
      for f in range(WS // L):
        ov = sc * (abuf[r, pl.ds(f * L, L)] + ubuf[r, pl.ds(f * L, L)])
        if z_ref is not None:
          ov = ov + zcbuf[r, pl.ds(f * L, L)]
        obuf[r, pl.ds(f * L, L)] = ov
      return c

    lax.fori_loop(0, CCH, rbody, 0)
    pltpu.sync_copy(obuf, dst_ref.at[pl.ds(dst_off + r0, CCH)])
    return carry

  lax.fori_loop(0, RPT // CCH, body, 0)


def _make_chain(K, cpt, with_z):
  def body(u0, pk_e, d2, dinv, *rest):
    if with_z:
      z0, out = rest[0], rest[1]
      scratches = rest[2:]
    else:
      z0 = None
      out = rest[0]
      scratches = rest[1:]
    if K == 1:
      (acc, pball, gidx0, sidx0, gidx1, sidx1, rb0, rb1, abuf, ubuf, obuf,
       scb, zcbuf, zbuf, sem0, sem1) = scratches
      ua = None
    else:
      (acc, ua, pball, gidx0, sidx0, gidx1, sidx1, rb0, rb1, abuf, ubuf,
       obuf, scb, zcbuf, zbuf, sem0, sem1) = scratches
    core = lax.axis_index("c")
    sub = lax.axis_index("s")
    base = core * R
    nrow0 = sub * RPT
    _fill(zbuf, CCH, WS, 0.0)
    pltpu.sync_copy(pk_e.at[pl.ds(sub * cpt, cpt)], pball.at[pl.ds(0, cpt)])
    dummy = u0.at[pl.ds(0, CH)]

    for i in range(RPT // CCH):
      pltpu.sync_copy(zbuf, acc.at[pl.ds(nrow0 + i * CCH, CCH)])

    if K == 1:
      plsc.subcore_barrier()
      _scatter_step(u0, cpt, acc, pball, gidx0, sidx0, gidx1, sidx1,
                    rb0, rb1, sem0, sem1, base, dummy)
      plsc.subcore_barrier()
      _combine(u0, out, z0, dinv, acc, abuf, ubuf, obuf, scb, zcbuf, zbuf,
               base, base, sub, False)
    else:
      pltpu.sync_copy(u0.at[pl.ds(base + nrow0, RPT)],
                      ua.at[pl.ds(nrow0, RPT)])

      def step(i, carry):
        plsc.subcore_barrier()
        _scatter_step(ua, cpt, acc, pball, gidx0, sidx0, gidx1, sidx1,
                      rb0, rb1, sem0, sem1, 0, dummy)
        plsc.subcore_barrier()
        _combine(ua, ua, None, d2, acc, abuf, ubuf, obuf, scb, zcbuf, zbuf,
                 0, 0, sub, True)
        return carry

      lax.fori_loop(0, K - 1, step, 0)
      plsc.subcore_barrier()
      _scatter_step(ua, cpt, acc, pball, gidx0, sidx0, gidx1, sidx1,
                    rb0, rb1, sem0, sem1, 0, dummy)
      plsc.subcore_barrier()
      _combine(ua, out, None, dinv, acc, abuf, ubuf, obuf, scb, zcbuf, zbuf,
               0, base, sub, False)

  out_t = jax.ShapeDtypeStruct((NC * R, WS), jnp.float32)
  scratch = [pltpu.VMEM_SHARED((R, WS), jnp.float32)]
  if K > 1:
    scratch += [pltpu.VMEM_SHARED((R, WS), jnp.float32)]
  scratch += [
      pltpu.VMEM((cpt, CH), jnp.int32),
      pltpu.VMEM((CH,), jnp.int32),
      pltpu.VMEM((CH,), jnp.int32),
      pltpu.VMEM((CH,), jnp.int32),
      pltpu.VMEM((CH,), jnp.int32),
      pltpu.VMEM((CH, WS), jnp.float32),
      pltpu.VMEM((CH, WS), jnp.float32),
      pltpu.VMEM((CCH, WS), jnp.float32),
      pltpu.VMEM((CCH, WS), jnp.float32),
      pltpu.VMEM((CCH, WS), jnp.float32),
      pltpu.VMEM((CCH, L), jnp.float32),
      pltpu.VMEM((CCH, WS), jnp.float32),
      pltpu.VMEM((CCH, WS), jnp.float32),
      pltpu.SemaphoreType.DMA,
      pltpu.SemaphoreType.DMA,
  ]
  return pl.kernel(body, out_type=out_t, mesh=_mesh(), scratch_types=scratch,
                   compiler_params=pltpu.CompilerParams(
                       use_tc_tiling_on_sc=False))


def _deg_body(pk_m, pk_k, deg_m, deg_k, acc, pball, dball, onesb, abuf, zbuf,
              sem):
  core = lax.axis_index("c")
  sub = lax.axis_index("s")
  _fill(zbuf, CCH, L, 0.0)
  _fill(onesb, CH, L, 1.0)

  def histo(pk_ref, cpt, deg_out):
    nrow0 = sub * RPT
    pltpu.sync_copy(pk_ref.at[pl.ds(sub * cpt, cpt)], pball.at[pl.ds(0, cpt)])
    for i in range(RPT // CCH):
      pltpu.sync_copy(zbuf, acc.at[pl.ds(nrow0 + i * CCH, CCH)])

    def prep(i, c):
      for j in range(CH // L):
        w = pball[i, pl.ds(j * L, L)]
        s = w & jnp.int32(0xFFFF)
        dd = w >> 16
        dball[i, pl.ds(j * L, L)] = jnp.where(s == dd, jnp.int32(DUM), dd)
      return c

    lax.fori_loop(0, cpt, prep, 0)
    plsc.subcore_barrier()

    def group(i, c):
      for j in range(8):
        pltpu.async_copy(onesb, acc.at[dball.at[8 * i + j]], sem, add=True)
      for j in range(8):
        pltpu.make_async_copy(deg_out.at[pl.ds(0, CH)], onesb, sem).wait()
      return c

    lax.fori_loop(0, cpt // 8, group, 0)
    plsc.subcore_barrier()

    def comb(i, c):
      r0 = nrow0 + i * CCH
      pltpu.sync_copy(acc.at[pl.ds(r0, CCH)], abuf)

      def rbody(r, cc):
        abuf[r, :] = abuf[r, :] + 1.0
        return cc

      lax.fori_loop(0, CCH, rbody, 0)
      pltpu.sync_copy(abuf, deg_out.at[pl.ds(r0, CCH)])
      return c

    lax.fori_loop(0, RPT // CCH, comb, 0)

  @pl.when(core == 0)
  def _():
    histo(pk_m, CPT_M, deg_m)

  @pl.when(core == 1)
  def _():
    histo(pk_k, CPT_K, deg_k)


@functools.cache
def _deg_kernel():
  return pl.kernel(
    _deg_body,
    out_type=(jax.ShapeDtypeStruct((R, L), jnp.float32),) * 2,
    mesh=_mesh(),
    scratch_types=[
        pltpu.VMEM_SHARED((R, L), jnp.float32),
        pltpu.VMEM((CPT_M, CH), jnp.int32),
        pltpu.VMEM((CPT_M, CH), jnp.int32),
        pltpu.VMEM((CH, L), jnp.float32),
        pltpu.VMEM((CCH, L), jnp.float32),
        pltpu.VMEM((CCH, L), jnp.float32),
        pltpu.SemaphoreType.DMA,
    ],
    compiler_params=pltpu.CompilerParams(use_tc_tiling_on_sc=False),
  )


def _norm_body(dm_ref, dk_ref, im_ref, qm_ref, ik_ref, qk_ref):
  dm = dm_ref[...]
  dk = dk_ref[...]
  im_ref[...] = lax.rsqrt(dm)
  qm_ref[...] = 1.0 / dm
  ik_ref[...] = lax.rsqrt(dk)
  qk_ref[...] = 1.0 / dk


def _norm(deg_m, deg_k):
  return pl.pallas_call(
      _norm_body,
      out_shape=(jax.ShapeDtypeStruct((R, L), jnp.float32),) * 4,
  )(deg_m, deg_k)


def _mm_body(x_ref, w_ref, b_ref, dm_ref, dk_ref, o_ref):
  y = jnp.dot(x_ref[...], w_ref[...], preferred_element_type=jnp.float32)
  y = y + b_ref[0:1, :]
  s = dm_ref[:, 0:1]
  sk = dk_ref[:, 0:1]
  o_ref[:, 0:384] = y[:, 0:384] * s
  o_ref[:, 384:512] = y[:, 384:512] * sk


_MMB = 1000


def _matmul(x, wt, b8, dm, dk):
  return pl.pallas_call(
      _mm_body,
      grid=(N // _MMB,),
      in_specs=[
          pl.BlockSpec((_MMB, D), lambda i: (i, 0)),
          pl.BlockSpec((D, 512), lambda i: (0, 0)),
          pl.BlockSpec((8, 512), lambda i: (0, 0)),
          pl.BlockSpec((_MMB, L), lambda i: (i, 0)),
          pl.BlockSpec((_MMB, L), lambda i: (i, 0)),
      ],
      out_specs=pl.BlockSpec((_MMB, 512), lambda i: (i, 0)),
      out_shape=jax.ShapeDtypeStruct((N, 512), jnp.float32),
  )(x, wt, b8, dm, dk)


def _pack_edges(ei, rows):
  e = ei.shape[1]
  tot = rows * CH
  src = jnp.pad(ei[0].astype(jnp.int32), (0, tot - e))
  pad_d = DUM + (jnp.arange(tot - e, dtype=jnp.int32) % (R - DUM))
  dst = jnp.concatenate([ei[1].astype(jnp.int32), pad_d])
  return (src | (dst << 16)).reshape(rows, CH)


def _to_sc(y):
  lo = jnp.pad(y[:, 0:WS], ((0, R - N), (0, 0)))
  hi = jnp.pad(y[:, WS:D], ((0, R - N), (0, 0)))
  return jnp.concatenate([lo, hi], axis=0)


def _from_sc(u):
  return jnp.concatenate([u[0:N], u[R:R + N]], axis=1)


_chain2 = functools.cache(lambda: _make_chain(2, CPT_M, False))
_chain10 = functools.cache(lambda: _make_chain(10, CPT_M, False))
_chain1z = functools.cache(lambda: _make_chain(1, CPT_K, True))


def kernel(x, edge_index, knn_graph, W_s1, b_s1, W_s2, b_s2, W_f2, b_f2):
  pk_m = _pack_edges(edge_index, ROWS_M)
  pk_k = _pack_edges(knn_graph, ROWS_K)

  deg_m, deg_k = _deg_kernel()(pk_m, pk_k)
  dinv_m, d2_m, dinv_k, d2_k = _norm(deg_m, deg_k)

  wt = jnp.concatenate([W_s1, W_s2, W_f2, W_f2], axis=0).T
  bcat = jnp.concatenate([b_s1, b_s2, b_f2, b_f2])
  b8 = jnp.broadcast_to(bcat[None, :], (8, 512))
  y = _matmul(x, wt, b8, dinv_m[0:N], dinv_k[0:N])

  u_s1 = _to_sc(y[:, 0:128])
  u_s2 = _to_sc(y[:, 128:256])
  u_f = _to_sc(y[:, 256:384])
  u_k = _to_sc(y[:, 384:512])

  h0u = _chain2()(u_s1, pk_m, d2_m, dinv_m)
  h1u = _chain10()(u_s2, pk_m, d2_m, dinv_m)
  z0u = _chain2()(u_f, pk_m, d2_m, dinv_m)
  z1u = _chain1z()(u_k, pk_k, d2_k, dinv_k, z0u)

  return (_from_sc(h0u), _from_sc(h1u), _from_sc(z0u), _from_sc(z1u))
